# Initial kernel scaffold; baseline (speedup 1.0000x reference)
import jax
import jax.numpy as jnp
from jax.experimental import pallas as pl


def kernel(src_table, tgt_table, src_indices, tgt_indices):
    raise NotImplementedError("write your pallas kernel here")


# speedup vs baseline: 3.0994x; 3.0994x over previous
import jax
import jax.numpy as jnp
from jax import lax
from jax.experimental import pallas as pl
from jax.experimental.pallas import tpu as pltpu
from jax.experimental.pallas import tpu_sc as plsc

B, L, EMBED = 4096, 50, 32
N = B * L
NC, NS = 2, 16
NW = NC * NS
PER_W = N // NW
CH = 3200
NCH = PER_W // CH


def _emb_body(src_hbm, tgt_hbm, sidx_hbm, tidx_hbm, src_out, tgt_out,
              idx_v, rows_v, sem):
    wid = lax.axis_index("s") * NC + lax.axis_index("c")
    base = wid * PER_W

    def run(table_hbm, idx_hbm, out_hbm):
        def chunk(c, carry):
            off = base + c * CH
            pltpu.sync_copy(idx_hbm.at[pl.ds(off, CH)], idx_v)
            pltpu.async_copy(table_hbm.at[idx_v], rows_v, sem).wait()
            pltpu.sync_copy(rows_v, out_hbm.at[pl.ds(off, CH)])
            return carry
        lax.fori_loop(0, NCH, chunk, 0)

    run(src_hbm, sidx_hbm, src_out)
    run(tgt_hbm, tidx_hbm, tgt_out)


def kernel(src_table, tgt_table, src_indices, tgt_indices):
    sidx = src_indices.reshape(-1).astype(jnp.int32)
    tidx = tgt_indices.reshape(-1).astype(jnp.int32)
    mesh = plsc.VectorSubcoreMesh(core_axis_name="c", subcore_axis_name="s")
    f = pl.kernel(
        _emb_body,
        mesh=mesh,
        out_type=(
            jax.ShapeDtypeStruct((N, EMBED), jnp.float32),
            jax.ShapeDtypeStruct((N, EMBED), jnp.float32),
        ),
        scratch_types=[
            pltpu.VMEM((CH,), jnp.int32),
            pltpu.VMEM((CH, EMBED), jnp.float32),
            pltpu.SemaphoreType.DMA,
        ],
        compiler_params=pltpu.CompilerParams(use_tc_tiling_on_sc=False),
    )
    src_out, tgt_out = f(src_table, tgt_table, sidx, tidx)
    return (src_out.reshape(B, L, EMBED), tgt_out.reshape(B, L, EMBED))


# speedup vs baseline: 3.1046x; 1.0017x over previous
import jax
import jax.numpy as jnp
from jax import lax
from jax.experimental import pallas as pl
from jax.experimental.pallas import tpu as pltpu
from jax.experimental.pallas import tpu_sc as plsc

B, L, EMBED = 4096, 50, 32
N = B * L
NC, NS = 2, 16
NW = NC * NS
PER_W = N // NW
CH = 1280
NCH = PER_W // CH
NB = 3


def _emb_body(src_hbm, tgt_hbm, sidx_hbm, tidx_hbm, src_out, tgt_out,
              *scratch):
    idx = scratch[0:NB]
    rows = scratch[NB:2 * NB]
    si = scratch[2 * NB:3 * NB]
    sg = scratch[3 * NB:4 * NB]
    ss = scratch[4 * NB:5 * NB]

    wid = lax.axis_index("s") * NC + lax.axis_index("c")
    base = wid * PER_W

    specs = []
    for tab, ih, oh in ((src_hbm, sidx_hbm, src_out),
                        (tgt_hbm, tidx_hbm, tgt_out)):
        for c in range(NCH):
            specs.append((tab, ih, oh, c * CH))
    n = len(specs)

    def idx_start(k):
        _, ih, _, off = specs[k]
        return pltpu.async_copy(ih.at[pl.ds(base + off, CH)], idx[k % NB],
                                si[k % NB])

    def gather_start(k):
        tab, _, _, _ = specs[k]
        return pltpu.async_copy(tab.at[idx[k % NB]], rows[k % NB], sg[k % NB])

    def store_start(k):
        _, _, oh, off = specs[k]
        return pltpu.async_copy(rows[k % NB], oh.at[pl.ds(base + off, CH)],
                                ss[k % NB])

    hi = [None] * n
    hg = [None] * n
    hs = [None] * n
    for k in range(min(NB, n)):
        hi[k] = idx_start(k)
    for k in range(n):
        hi[k].wait()
        if k >= NB:
            hs[k - NB].wait()
        hg[k] = gather_start(k)
        if k >= 1:
            hg[k - 1].wait()
            hs[k - 1] = store_start(k - 1)
            if k - 1 + NB < n:
                hi[k - 1 + NB] = idx_start(k - 1 + NB)
    hg[n - 1].wait()
    hs[n - 1] = store_start(n - 1)
    for k in range(max(0, n - NB), n):
        hs[k].wait()


def kernel(src_table, tgt_table, src_indices, tgt_indices):
    sidx = src_indices.reshape(-1).astype(jnp.int32)
    tidx = tgt_indices.reshape(-1).astype(jnp.int32)
    mesh = plsc.VectorSubcoreMesh(core_axis_name="c", subcore_axis_name="s")
    scratch = ([pltpu.VMEM((CH,), jnp.int32) for _ in range(NB)]
               + [pltpu.VMEM((CH, EMBED), jnp.float32) for _ in range(NB)]
               + [pltpu.SemaphoreType.DMA for _ in range(3 * NB)])
    f = pl.kernel(
        _emb_body,
        mesh=mesh,
        out_type=(
            jax.ShapeDtypeStruct((N, EMBED), jnp.float32),
            jax.ShapeDtypeStruct((N, EMBED), jnp.float32),
        ),
        scratch_types=scratch,
        compiler_params=pltpu.CompilerParams(use_tc_tiling_on_sc=False),
    )
    src_out, tgt_out = f(src_table, tgt_table, sidx, tidx)
    return (src_out.reshape(B, L, EMBED), tgt_out.reshape(B, L, EMBED))


# speedup vs baseline: 3.2270x; 1.0394x over previous
import jax
import jax.numpy as jnp
from jax import lax
from jax.experimental import pallas as pl
from jax.experimental.layout import Format, Layout, with_layout_constraint
from jax.experimental.pallas import tpu as pltpu
from jax.experimental.pallas import tpu_sc as plsc

B, L, EMBED = 4096, 50, 32
N = B * L
NC, NS = 2, 16
NW = NC * NS
PER_W = N // NW
CH = 1280
NCH = PER_W // CH
NB = 3


def _emb_body(src_hbm, tgt_hbm, sidx_hbm, tidx_hbm, src_out, tgt_out,
              *scratch):
    idx = scratch[0:NB]
    rows = scratch[NB:2 * NB]
    si = scratch[2 * NB:3 * NB]
    sg = scratch[3 * NB:4 * NB]
    ss = scratch[4 * NB:5 * NB]

    wid = lax.axis_index("s") * NC + lax.axis_index("c")
    base = wid * PER_W

    specs = []
    for tab, ih, oh in ((src_hbm, sidx_hbm, src_out),
                        (tgt_hbm, tidx_hbm, tgt_out)):
        for c in range(NCH):
            specs.append((tab, ih, oh, c * CH))
    n = len(specs)

    def idx_start(k):
        _, ih, _, off = specs[k]
        return pltpu.async_copy(ih.at[pl.ds(base + off, CH)], idx[k % NB],
                                si[k % NB])

    def gather_start(k):
        tab, _, _, _ = specs[k]
        return pltpu.async_copy(tab.at[idx[k % NB]], rows[k % NB], sg[k % NB])

    def store_start(k):
        _, _, oh, off = specs[k]
        return pltpu.async_copy(rows[k % NB], oh.at[pl.ds(base + off, CH)],
                                ss[k % NB])

    hi = [None] * n
    hg = [None] * n
    hs = [None] * n
    for k in range(min(NB, n)):
        hi[k] = idx_start(k)
    for k in range(n):
        hi[k].wait()
        if k >= NB:
            hs[k - NB].wait()
        hg[k] = gather_start(k)
        if k >= 1:
            hg[k - 1].wait()
            hs[k - 1] = store_start(k - 1)
            if k - 1 + NB < n:
                hi[k - 1 + NB] = idx_start(k - 1 + NB)
    hg[n - 1].wait()
    hs[n - 1] = store_start(n - 1)
    for k in range(max(0, n - NB), n):
        hs[k].wait()


def kernel(src_table, tgt_table, src_indices, tgt_indices):
    lin2 = Layout(major_to_minor=(0, 1), tiling=())
    src_table, tgt_table = with_layout_constraint(
        (src_table, tgt_table), (lin2, lin2))
    lin1 = Layout(major_to_minor=(0,), tiling=())
    sidx = src_indices.reshape(-1).astype(jnp.int32)
    tidx = tgt_indices.reshape(-1).astype(jnp.int32)
    sidx, tidx = with_layout_constraint((sidx, tidx), (lin1, lin1))
    mesh = plsc.VectorSubcoreMesh(core_axis_name="c", subcore_axis_name="s")
    scratch = ([pltpu.VMEM((CH,), jnp.int32) for _ in range(NB)]
               + [pltpu.VMEM((CH, EMBED), jnp.float32) for _ in range(NB)]
               + [pltpu.SemaphoreType.DMA for _ in range(3 * NB)])
    f = pl.kernel(
        _emb_body,
        mesh=mesh,
        out_type=(
            jax.ShapeDtypeStruct((N, EMBED), jnp.float32),
            jax.ShapeDtypeStruct((N, EMBED), jnp.float32),
        ),
        scratch_types=scratch,
        compiler_params=pltpu.CompilerParams(use_tc_tiling_on_sc=False),
    )
    src_out, tgt_out = f(src_table, tgt_table, sidx, tidx)
    lout = Layout(major_to_minor=(0, 1), tiling=())
    src_out, tgt_out = with_layout_constraint((src_out, tgt_out),
                                              (lout, lout))
    return (src_out.reshape(B, L, EMBED), tgt_out.reshape(B, L, EMBED))


# speedup vs baseline: 5.2291x; 1.6204x over previous
import jax
import jax.numpy as jnp
from jax import lax
from jax.experimental import pallas as pl
from jax.experimental.pallas import tpu as pltpu
from jax.experimental.pallas import tpu_sc as plsc

B, L, EMBED = 4096, 50, 32
N = B * L
NC, NS = 2, 16
NW = NC * NS
BPW = B // NW
CB = 32
CH = CB * L
NCH = BPW // CB
NB = 2


def _emb_body(src_hbm, tgt_hbm, sidx_hbm, tidx_hbm, src_out, tgt_out,
              *scratch):
    idx = scratch[0:NB]
    rows = scratch[NB:2 * NB]
    si = scratch[2 * NB:3 * NB]
    sg = scratch[3 * NB:4 * NB]
    ss = scratch[4 * NB:5 * NB]

    wid = lax.axis_index("s") * NC + lax.axis_index("c")
    base_b = wid * BPW

    specs = []
    for tab, ih, oh in ((src_hbm, sidx_hbm, src_out),
                        (tgt_hbm, tidx_hbm, tgt_out)):
        for c in range(NCH):
            specs.append((tab, ih, oh, c * CB))
    n = len(specs)

    def idx_start(k):
        _, ih, _, off = specs[k]
        return pltpu.async_copy(
            ih.at[pl.ds((base_b + off) * L, CH)], idx[k % NB], si[k % NB])

    def gather_start(k):
        tab, _, _, _ = specs[k]
        return pltpu.async_copy(tab.at[idx[k % NB]], rows[k % NB], sg[k % NB])

    def store_start(k):
        _, _, oh, off = specs[k]
        b = k % NB
        return [pltpu.async_copy(rows[b].at[pl.ds(j * L, L)],
                                 oh.at[base_b + off + j], ss[b])
                for j in range(CB)]

    def store_wait(hs_k):
        for h in hs_k:
            h.wait()

    hi = [None] * n
    hg = [None] * n
    hs = [None] * n
    for k in range(min(NB, n)):
        hi[k] = idx_start(k)
    for k in range(n):
        hi[k].wait()
        if k >= NB:
            store_wait(hs[k - NB])
        hg[k] = gather_start(k)
        if k >= 1:
            hg[k - 1].wait()
            hs[k - 1] = store_start(k - 1)
            if k - 1 + NB < n:
                hi[k - 1 + NB] = idx_start(k - 1 + NB)
    hg[n - 1].wait()
    hs[n - 1] = store_start(n - 1)
    for k in range(max(0, n - NB), n):
        store_wait(hs[k])


def kernel(src_table, tgt_table, src_indices, tgt_indices):
    sidx = src_indices.reshape(-1).astype(jnp.int32)
    tidx = tgt_indices.reshape(-1).astype(jnp.int32)
    mesh = plsc.VectorSubcoreMesh(core_axis_name="c", subcore_axis_name="s")
    scratch = ([pltpu.VMEM((CH,), jnp.int32) for _ in range(NB)]
               + [pltpu.VMEM((CH, EMBED), jnp.float32) for _ in range(NB)]
               + [pltpu.SemaphoreType.DMA for _ in range(3 * NB)])
    f = pl.kernel(
        _emb_body,
        mesh=mesh,
        out_type=(
            jax.ShapeDtypeStruct((B, L, EMBED), jnp.float32),
            jax.ShapeDtypeStruct((B, L, EMBED), jnp.float32),
        ),
        scratch_types=scratch,
        compiler_params=pltpu.CompilerParams(use_tc_tiling_on_sc=False),
    )
    src_out, tgt_out = f(src_table, tgt_table, sidx, tidx)
    return (src_out, tgt_out)
